# baseline (speedup 1.0000x reference)
import functools

import jax
import jax.numpy as jnp
from jax import lax
from jax.experimental import pallas as pl
from jax.experimental.pallas import tpu as pltpu
from jax.experimental.pallas import tpu_sc as plsc

N = 10000
E = 320000
D = 128

NC = 2
NS = 16
NW = NC * NS

CH = 128
CH_A = 98
CH_B = 59
NCHUNK = max(CH_A, CH_B)
CAP_A = NS * CH_A * CH
CAP_B = NS * CH_B * CH

ROWS = 640
NPAD = ROWS * NS

_mesh = plsc.VectorSubcoreMesh(core_axis_name="c", subcore_axis_name="s",
                               num_cores=NC, num_subcores=NS)


@functools.partial(
    pl.kernel,
    out_type=jax.ShapeDtypeStruct((NC, NPAD, D), jnp.float32),
    mesh=_mesh,
    scratch_types=[
        pltpu.VMEM((NCHUNK, CH), jnp.int32),
        pltpu.VMEM((NCHUNK, CH), jnp.int32),
        pltpu.VMEM((CH, D), jnp.float32),
        pltpu.VMEM_SHARED((NPAD, D), jnp.float32),
        pltpu.SemaphoreType.DMA,
    ],
)
def _sc_segment_sum(x_hbm, src_hbm, dst_hbm, zeros_hbm, out_hbm,
                    src_v, dst_v, rows_v, agg_sh, sem):
    c = lax.axis_index("c")
    s = lax.axis_index("s")
    wid = c * NS + s
    nch = jnp.where(c == 0, CH_A, CH_B)

    pltpu.sync_copy(zeros_hbm, agg_sh.at[pl.ds(s * ROWS, ROWS)])
    pltpu.sync_copy(src_hbm.at[wid], src_v)
    pltpu.sync_copy(dst_hbm.at[wid], dst_v)
    plsc.subcore_barrier()

    def step(j, carry):
        pltpu.async_copy(x_hbm.at[src_v.at[j]], rows_v, sem).wait()
        pltpu.sync_copy(rows_v, agg_sh.at[dst_v.at[j]], add=True)
        return carry

    lax.fori_loop(0, nch, step, 0)
    plsc.subcore_barrier()
    pltpu.sync_copy(agg_sh.at[pl.ds(s * ROWS, ROWS)],
                    out_hbm.at[c, pl.ds(s * ROWS, ROWS)])


_R = 1000


def _dense_body(act, part_ref, x_ref, wr_ref, wt_ref, b_ref, o_ref):
    agg = part_ref[0] + part_ref[1]
    dn = (((1,), (1,)), ((), ()))
    v = lax.dot_general(agg, wr_ref[...], dn,
                        preferred_element_type=jnp.float32)
    v = v + lax.dot_general(x_ref[...], wt_ref[...], dn,
                            preferred_element_type=jnp.float32)
    v = v + b_ref[...]
    if act == "elu":
        o_ref[...] = jnp.where(v > 0, v, jnp.exp(v) - 1.0)
    else:
        o_ref[...] = 1.0 / (1.0 + jnp.exp(-v))


def _dense(part, x, w_rel, b_rel, w_root, act):
    return pl.pallas_call(
        functools.partial(_dense_body, act),
        grid=(N // _R,),
        in_specs=[
            pl.BlockSpec((NC, _R, D), lambda i: (0, i, 0)),
            pl.BlockSpec((_R, D), lambda i: (i, 0)),
            pl.BlockSpec((D, D), lambda i: (0, 0)),
            pl.BlockSpec((D, D), lambda i: (0, 0)),
            pl.BlockSpec((1, D), lambda i: (0, 0)),
        ],
        out_specs=pl.BlockSpec((_R, D), lambda i: (i, 0)),
        out_shape=jax.ShapeDtypeStruct((N, D), jnp.float32),
    )(part, x, w_rel, w_root, b_rel.reshape(1, D))


def _slab(v, n_chunks, pad_value):
    cap = NS * n_chunks * CH
    v = jnp.concatenate(
        [v, jnp.full((cap - v.shape[0],), pad_value, jnp.int32)])
    v = v.reshape(NS, n_chunks, CH)
    return jnp.pad(v, ((0, 0), (0, NCHUNK - n_chunks), (0, 0)))


def kernel(graph, edge_index, W_rel1, b_rel1, W_root1,
           W_rel2, b_rel2, W_root2, W_rel3, b_rel3, W_root3):
    src = edge_index[0].astype(jnp.int32)
    dst = edge_index[1].astype(jnp.int32)
    src_p = jnp.concatenate(
        [_slab(src[:CAP_A], CH_A, 0), _slab(src[CAP_A:], CH_B, 0)])
    dst_p = jnp.concatenate(
        [_slab(dst[:CAP_A], CH_A, N), _slab(dst[CAP_A:], CH_B, N)])
    zeros = jnp.zeros((ROWS, D), jnp.float32)

    x = graph
    outs = []
    for w_rel, b_rel, w_root, act in (
        (W_rel1, b_rel1, W_root1, "elu"),
        (W_rel2, b_rel2, W_root2, "elu"),
        (W_rel3, b_rel3, W_root3, "sigmoid"),
    ):
        part = _sc_segment_sum(x, src_p, dst_p, zeros)
        x = _dense(part, x, w_rel, b_rel, w_root, act)
        outs.append(x)
    return tuple(outs)


# speedup vs baseline: 1.0268x; 1.0268x over previous
import functools

import jax
import jax.numpy as jnp
from jax import lax
from jax.experimental import pallas as pl
from jax.experimental.pallas import tpu as pltpu
from jax.experimental.pallas import tpu_sc as plsc

N = 10000
E = 320000
D = 128

NC = 2
NS = 16
NW = NC * NS

CH = 128
CH_A = 94
CH_B = 63
NCHUNK = max(CH_A, CH_B)
CAP_A = NS * CH_A * CH
CAP_B = NS * CH_B * CH

ROWS = 640
NPAD = ROWS * NS

_mesh = plsc.VectorSubcoreMesh(core_axis_name="c", subcore_axis_name="s",
                               num_cores=NC, num_subcores=NS)


@functools.partial(
    pl.kernel,
    out_type=jax.ShapeDtypeStruct((NC, NPAD, D), jnp.float32),
    mesh=_mesh,
    scratch_types=[
        pltpu.VMEM((NCHUNK, CH), jnp.int32),
        pltpu.VMEM((NCHUNK, CH), jnp.int32),
        pltpu.VMEM((CH, D), jnp.float32),
        pltpu.VMEM_SHARED((NPAD, D), jnp.float32),
        pltpu.SemaphoreType.DMA,
    ],
)
def _sc_segment_sum(x_hbm, src_hbm, dst_hbm, zeros_hbm, out_hbm,
                    src_v, dst_v, rows_v, agg_sh, sem):
    c = lax.axis_index("c")
    s = lax.axis_index("s")
    wid = c * NS + s
    nch = jnp.where(c == 0, CH_A, CH_B)

    pltpu.sync_copy(zeros_hbm, agg_sh.at[pl.ds(s * ROWS, ROWS)])
    pltpu.sync_copy(src_hbm.at[wid], src_v)
    pltpu.sync_copy(dst_hbm.at[wid], dst_v)
    plsc.subcore_barrier()

    def step(j, carry):
        pltpu.async_copy(x_hbm.at[src_v.at[j]], rows_v, sem).wait()
        pltpu.sync_copy(rows_v, agg_sh.at[dst_v.at[j]], add=True)
        return carry

    lax.fori_loop(0, nch, step, 0)
    plsc.subcore_barrier()
    pltpu.sync_copy(agg_sh.at[pl.ds(s * ROWS, ROWS)],
                    out_hbm.at[c, pl.ds(s * ROWS, ROWS)])


_R = 1000


def _dense_body(act, part_ref, x_ref, wr_ref, wt_ref, b_ref, o_ref):
    agg = part_ref[0] + part_ref[1]
    dn = (((1,), (1,)), ((), ()))
    v = lax.dot_general(agg, wr_ref[...], dn,
                        preferred_element_type=jnp.float32)
    v = v + lax.dot_general(x_ref[...], wt_ref[...], dn,
                            preferred_element_type=jnp.float32)
    v = v + b_ref[...]
    if act == "elu":
        o_ref[...] = jnp.where(v > 0, v, jnp.exp(v) - 1.0)
    else:
        o_ref[...] = 1.0 / (1.0 + jnp.exp(-v))


def _dense(part, x, w_rel, b_rel, w_root, act):
    return pl.pallas_call(
        functools.partial(_dense_body, act),
        grid=(N // _R,),
        in_specs=[
            pl.BlockSpec((NC, _R, D), lambda i: (0, i, 0)),
            pl.BlockSpec((_R, D), lambda i: (i, 0)),
            pl.BlockSpec((D, D), lambda i: (0, 0)),
            pl.BlockSpec((D, D), lambda i: (0, 0)),
            pl.BlockSpec((1, D), lambda i: (0, 0)),
        ],
        out_specs=pl.BlockSpec((_R, D), lambda i: (i, 0)),
        out_shape=jax.ShapeDtypeStruct((N, D), jnp.float32),
    )(part, x, w_rel, w_root, b_rel.reshape(1, D))


def _slab(v, n_chunks, pad_value):
    cap = NS * n_chunks * CH
    v = jnp.concatenate(
        [v, jnp.full((cap - v.shape[0],), pad_value, jnp.int32)])
    v = v.reshape(NS, n_chunks, CH)
    return jnp.pad(v, ((0, 0), (0, NCHUNK - n_chunks), (0, 0)))


def kernel(graph, edge_index, W_rel1, b_rel1, W_root1,
           W_rel2, b_rel2, W_root2, W_rel3, b_rel3, W_root3):
    src = edge_index[0].astype(jnp.int32)
    dst = edge_index[1].astype(jnp.int32)
    src_p = jnp.concatenate(
        [_slab(src[:CAP_A], CH_A, 0), _slab(src[CAP_A:], CH_B, 0)])
    dst_p = jnp.concatenate(
        [_slab(dst[:CAP_A], CH_A, N), _slab(dst[CAP_A:], CH_B, N)])
    zeros = jnp.zeros((ROWS, D), jnp.float32)

    x = graph
    outs = []
    for w_rel, b_rel, w_root, act in (
        (W_rel1, b_rel1, W_root1, "elu"),
        (W_rel2, b_rel2, W_root2, "elu"),
        (W_rel3, b_rel3, W_root3, "sigmoid"),
    ):
        part = _sc_segment_sum(x, src_p, dst_p, zeros)
        x = _dense(part, x, w_rel, b_rel, w_root, act)
        outs.append(x)
    return tuple(outs)
